# Initial kernel scaffold; baseline (speedup 1.0000x reference)
import jax
import jax.numpy as jnp
from jax.experimental import pallas as pl


def kernel(x, edge_index, edge_attr, u, batch, W1, b1, W2, b2, Wa, ba):
    raise NotImplementedError("write your pallas kernel here")


# speedup vs baseline: 13.7164x; 13.7164x over previous
import jax
import jax.numpy as jnp
from jax import lax
from jax.experimental import pallas as pl
from jax.experimental.pallas import tpu as pltpu

_NUM_GRAPHS = 64
_BE = 2560


def _body(src_ref, attr_ref, batch_ref, x_ref, u_ref, w1u_ref, w1n_ref,
          w1e_ref, b1_ref, w2_ref, b2_ref, wa_ref, ba_ref, out_ref,
          nsum, ncnt, blo, bhi, eacc, ecnt):
    i = pl.program_id(0)
    nb = pl.num_programs(0)

    @pl.when(i == 0)
    def _init():
        b = batch_ref[...]
        gcol = lax.broadcasted_iota(jnp.int32, (_NUM_GRAPHS, 1), 0)
        lt = (b < gcol).astype(jnp.int32)
        le = (b <= gcol).astype(jnp.int32)
        blo[...] = jnp.sum(lt, axis=1, keepdims=True)
        bhi[...] = jnp.sum(le, axis=1, keepdims=True)
        onehot = (le - lt).astype(jnp.float32)
        nsum[...] = jnp.dot(onehot, x_ref[...],
                            preferred_element_type=jnp.float32)
        ncnt[...] = jnp.sum(onehot, axis=1, keepdims=True)
        eacc[...] = jnp.zeros_like(eacc)
        ecnt[...] = jnp.zeros_like(ecnt)

    src = src_ref[0]
    m = ((src >= blo[...]) & (src < bhi[...])).astype(jnp.float32)
    eacc[...] += jnp.dot(m, attr_ref[...], preferred_element_type=jnp.float32)
    ecnt[...] += jnp.sum(m, axis=1, keepdims=True)

    @pl.when(i == nb - 1)
    def _finish():
        nmean = nsum[...] / jnp.maximum(ncnt[...], 1.0)
        emean = eacc[...] / jnp.maximum(ecnt[...], 1.0)
        h = (jnp.dot(u_ref[...], w1u_ref[...],
                     preferred_element_type=jnp.float32)
             + jnp.dot(nmean, w1n_ref[...],
                       preferred_element_type=jnp.float32)
             + jnp.dot(emean, w1e_ref[...],
                       preferred_element_type=jnp.float32)
             + b1_ref[...])
        h = jnp.maximum(h, 0.0)
        g = jnp.dot(h, w2_ref[...], preferred_element_type=jnp.float32) \
            + b2_ref[...]
        a = jax.nn.sigmoid(jnp.dot(g, wa_ref[...],
                                   preferred_element_type=jnp.float32)
                           + ba_ref[...])
        out_ref[...] = g * a


def kernel(x, edge_index, edge_attr, u, batch, W1, b1, W2, b2, Wa, ba):
    N, node_dim = x.shape
    E, edge_dim = edge_attr.shape
    global_dim = u.shape[1]
    hidden_dim = W1.shape[1]
    nb = E // _BE
    assert nb * _BE == E

    src = edge_index[0].astype(jnp.int32).reshape(nb, 1, _BE)
    batch2d = batch.astype(jnp.int32).reshape(1, N)
    w1u = W1[:global_dim]
    w1n = W1[global_dim:global_dim + node_dim]
    w1e = W1[global_dim + node_dim:]
    b1r = b1.reshape(1, hidden_dim)
    b2r = b2.reshape(1, global_dim)
    bar = ba.reshape(1, 1)

    in_specs = [
            pl.BlockSpec((1, 1, _BE), lambda i: (i, 0, 0)),
            pl.BlockSpec((_BE, edge_dim), lambda i: (i, 0)),
            pl.BlockSpec((1, N), lambda i: (0, 0)),
            pl.BlockSpec((N, node_dim), lambda i: (0, 0)),
            pl.BlockSpec((_NUM_GRAPHS, global_dim), lambda i: (0, 0)),
            pl.BlockSpec((global_dim, hidden_dim), lambda i: (0, 0)),
            pl.BlockSpec((node_dim, hidden_dim), lambda i: (0, 0)),
            pl.BlockSpec((edge_dim, hidden_dim), lambda i: (0, 0)),
            pl.BlockSpec((1, hidden_dim), lambda i: (0, 0)),
            pl.BlockSpec((hidden_dim, global_dim), lambda i: (0, 0)),
            pl.BlockSpec((1, global_dim), lambda i: (0, 0)),
            pl.BlockSpec((global_dim, 1), lambda i: (0, 0)),
            pl.BlockSpec((1, 1), lambda i: (0, 0)),
    ]

    return pl.pallas_call(
        _body,
        grid=(nb,),
        in_specs=in_specs,
        out_specs=pl.BlockSpec((_NUM_GRAPHS, global_dim), lambda i: (0, 0)),
        out_shape=jax.ShapeDtypeStruct((_NUM_GRAPHS, global_dim), jnp.float32),
        scratch_shapes=[
            pltpu.VMEM((_NUM_GRAPHS, node_dim), jnp.float32),
            pltpu.VMEM((_NUM_GRAPHS, 1), jnp.float32),
            pltpu.VMEM((_NUM_GRAPHS, 1), jnp.int32),
            pltpu.VMEM((_NUM_GRAPHS, 1), jnp.int32),
            pltpu.VMEM((_NUM_GRAPHS, edge_dim), jnp.float32),
            pltpu.VMEM((_NUM_GRAPHS, 1), jnp.float32),
        ],
        compiler_params=pltpu.CompilerParams(
            dimension_semantics=("arbitrary",),
        ),
    )(src, edge_attr, batch2d, x, u, w1u, w1n, w1e, b1r, W2, b2r, Wa, bar)
